# baseline (speedup 1.0000x reference)
import jax
import jax.numpy as jnp
from jax.experimental import pallas as pl

BLOCK_N = 512
_E = 16


def _split_hi_lo(a):
    hi_f32 = jax.lax.reduce_precision(a, exponent_bits=8, mantissa_bits=7)
    return hi_f32.astype(jnp.bfloat16), (a - hi_f32).astype(jnp.bfloat16)


def _assign_kernel(a_ref, b_ref, out_ref):
    r2 = jnp.dot(a_ref[...], b_ref[...],
                 preferred_element_type=jnp.float32)
    out_ref[...] = jnp.argmin(r2, axis=1).astype(jnp.int32)


def kernel(x, centroids):
    n, d = x.shape
    k = centroids.shape[0]
    ct = centroids.T
    cth, ctl = _split_hi_lo(ct)
    xh, xl = _split_hi_lo(x)
    cn2 = jnp.sum(ct * ct, axis=0)
    cn2h_f32 = jax.lax.reduce_precision(cn2, exponent_bits=8, mantissa_bits=7)
    cn2h = cn2h_f32.astype(jnp.bfloat16)
    cn2l = (cn2 - cn2h_f32).astype(jnp.bfloat16)

    e = jnp.zeros((n, _E), jnp.bfloat16).at[:, 0].set(1).at[:, 1].set(1)
    a_mat = jnp.concatenate([xh, xh, xl, e], axis=1)
    b_bias = jnp.zeros((_E, k), jnp.bfloat16).at[0].set(cn2h).at[1].set(cn2l)
    b_mat = jnp.concatenate(
        [-2.0 * cth, -2.0 * ctl, -2.0 * cth, b_bias], axis=0)

    dc = 3 * d + _E
    grid = (n // BLOCK_N,)
    assignments = pl.pallas_call(
        _assign_kernel,
        grid=grid,
        in_specs=[
            pl.BlockSpec((BLOCK_N, dc), lambda i: (i, 0)),
            pl.BlockSpec((dc, k), lambda i: (0, 0)),
        ],
        out_specs=pl.BlockSpec((BLOCK_N,), lambda i: (i,)),
        out_shape=jax.ShapeDtypeStruct((n,), jnp.int32),
    )(a_mat, b_mat)
    return (centroids[None, :, :], assignments)


# speedup vs baseline: 1.7650x; 1.7650x over previous
import jax
import jax.numpy as jnp
from jax.experimental import pallas as pl

BLOCK_N = 512
_SCALE = 65536.0
_CLIP = 31.5


def _split_hi_lo(a):
    hi_f32 = jax.lax.reduce_precision(a, exponent_bits=8, mantissa_bits=7)
    return hi_f32.astype(jnp.bfloat16), (a - hi_f32).astype(jnp.bfloat16)


def _assign_kernel(xh_ref, xl_ref, cth_ref, ctl_ref, ct_ref, out_ref):
    xh = xh_ref[...]
    xl = xl_ref[...]
    cth = cth_ref[...]
    ctl = ctl_ref[...]
    dot = lambda a, b: jnp.dot(a, b, preferred_element_type=jnp.float32)
    ct = ct_ref[...]
    cn2 = jnp.sum(ct * ct, axis=0)
    r2 = cn2[None, :] + (dot(xh, cth) + (dot(xh, ctl) + dot(xl, cth)))
    out_ref[...] = jnp.argmin(r2, axis=1).astype(jnp.int32)


def kernel(x, centroids):
    n, d = x.shape
    k = centroids.shape[0]
    ct = centroids.T
    cth, ctl = _split_hi_lo(ct)
    cth = -2.0 * cth
    ctl = -2.0 * ctl
    xh, xl = _split_hi_lo(x)
    grid = (n // BLOCK_N,)
    assignments = pl.pallas_call(
        _assign_kernel,
        grid=grid,
        in_specs=[
            pl.BlockSpec((BLOCK_N, d), lambda i: (i, 0)),
            pl.BlockSpec((BLOCK_N, d), lambda i: (i, 0)),
            pl.BlockSpec((d, k), lambda i: (0, 0)),
            pl.BlockSpec((d, k), lambda i: (0, 0)),
            pl.BlockSpec((d, k), lambda i: (0, 0)),
        ],
        out_specs=pl.BlockSpec((BLOCK_N,), lambda i: (i,)),
        out_shape=jax.ShapeDtypeStruct((n,), jnp.int32),
    )(xh, xl, cth, ctl, ct)
    return (centroids[None, :, :], assignments)


# speedup vs baseline: 2.1917x; 1.2418x over previous
import jax
import jax.numpy as jnp
from jax.experimental import pallas as pl
from jax.experimental.pallas import tpu as pltpu

BLOCK_N = 512


def _assign_kernel(x_ref, cth_ref, ctl_ref, ct_ref, out_ref, cn2_ref):
    @pl.when(pl.program_id(0) == 0)
    def _():
        ct = ct_ref[...]
        cn2_ref[...] = jnp.sum(ct * ct, axis=0, keepdims=True)

    x_blk = x_ref[...]
    bits = jax.lax.bitcast_convert_type(x_blk, jnp.int32)
    xh_f32 = jax.lax.bitcast_convert_type(
        bits & jnp.int32(-65536), jnp.float32)
    xh = xh_f32.astype(jnp.bfloat16)
    xl = (x_blk - xh_f32).astype(jnp.bfloat16)
    cth = cth_ref[...]
    ctl = ctl_ref[...]
    dot = lambda a, b: jnp.dot(a, b, preferred_element_type=jnp.float32)
    r2 = cn2_ref[...] + (dot(xh, cth) + (dot(xh, ctl) + dot(xl, cth)))
    out_ref[...] = jnp.argmin(r2, axis=1).astype(jnp.int32)


def _split_hi_lo(a):
    hi_f32 = jax.lax.reduce_precision(a, exponent_bits=8, mantissa_bits=7)
    return hi_f32.astype(jnp.bfloat16), (a - hi_f32).astype(jnp.bfloat16)


def kernel(x, centroids):
    n, d = x.shape
    k = centroids.shape[0]
    ct = centroids.T
    cth, ctl = _split_hi_lo(ct)
    cth = -2.0 * cth
    ctl = -2.0 * ctl
    grid = (n // BLOCK_N,)
    assignments = pl.pallas_call(
        _assign_kernel,
        grid=grid,
        in_specs=[
            pl.BlockSpec((BLOCK_N, d), lambda i: (i, 0)),
            pl.BlockSpec((d, k), lambda i: (0, 0)),
            pl.BlockSpec((d, k), lambda i: (0, 0)),
            pl.BlockSpec((d, k), lambda i: (0, 0)),
        ],
        out_specs=pl.BlockSpec((BLOCK_N,), lambda i: (i,)),
        out_shape=jax.ShapeDtypeStruct((n,), jnp.int32),
        scratch_shapes=[pltpu.VMEM((1, k), jnp.float32)],
    )(x, cth, ctl, ct)
    return (centroids[None, :, :], assignments)
